# baseline (speedup 1.0000x reference)
import jax
import jax.numpy as jnp
from jax.experimental import pallas as pl

_BATCH = 128
_VOCAB = 200
_DIM = 128
_BBLK = 16


def _bcast_kernel(w_ref, out_ref):
    out_ref[...] = jnp.broadcast_to(w_ref[...][None, :, :],
                                    (_BBLK, _VOCAB, _DIM))


def kernel(x, pe_weight):
    del x
    return pl.pallas_call(
        _bcast_kernel,
        grid=(_BATCH // _BBLK,),
        in_specs=[pl.BlockSpec((_VOCAB, _DIM), lambda i: (0, 0))],
        out_specs=pl.BlockSpec((_BBLK, _VOCAB, _DIM), lambda i: (i, 0, 0)),
        out_shape=jax.ShapeDtypeStruct((_BATCH, _VOCAB, _DIM), jnp.float32),
    )(pe_weight)


# speedup vs baseline: 1.1708x; 1.1708x over previous
import jax
import jax.numpy as jnp
from jax.experimental import pallas as pl
from jax.experimental.pallas import tpu as pltpu

_BATCH = 128
_VOCAB = 200
_DIM = 128
_R = 16
_NDMA = _BATCH // _R


def _copy_kernel(w_ref, out_ref, buf_ref, sem):
    buf_ref[...] = jnp.broadcast_to(w_ref[...][None, :, :],
                                    (_R, _VOCAB, _DIM))
    for i in range(_NDMA):
        pltpu.make_async_copy(
            buf_ref, out_ref.at[pl.ds(i * _R, _R)], sem.at[i]).start()
    for i in range(_NDMA):
        pltpu.make_async_copy(
            buf_ref, out_ref.at[pl.ds(i * _R, _R)], sem.at[i]).wait()


def kernel(x, pe_weight):
    del x
    return pl.pallas_call(
        _copy_kernel,
        in_specs=[pl.BlockSpec(memory_space=pltpu.MemorySpace.VMEM)],
        out_specs=pl.BlockSpec(memory_space=pltpu.MemorySpace.HBM),
        out_shape=jax.ShapeDtypeStruct((_BATCH, _VOCAB, _DIM), jnp.float32),
        scratch_shapes=[
            pltpu.VMEM((_R, _VOCAB, _DIM), jnp.float32),
            pltpu.SemaphoreType.DMA((_NDMA,)),
        ],
    )(pe_weight)
